# Initial kernel scaffold; baseline (speedup 1.0000x reference)
import jax
import jax.numpy as jnp
from jax.experimental import pallas as pl


def kernel(x, batch):
    raise NotImplementedError("write your pallas kernel here")


# speedup vs baseline: 2.8936x; 2.8936x over previous
import functools

import jax
import jax.numpy as jnp
from jax import lax
from jax.experimental import pallas as pl
from jax.experimental.pallas import tpu as pltpu
from jax.experimental.pallas import tpu_sc as plsc

N_ROWS = 50000
D = 256
NSEG = 128
CHUNK = 80
NBLK = N_ROWS // CHUNK
NC = 2
NS = 16
NW = NC * NS
SUMW = NSEG * D
CNTW = NSEG * 16


def _sc_segment_sums(x, batch, zsum, zcnt):
  mesh = plsc.VectorSubcoreMesh(core_axis_name="c", subcore_axis_name="s")

  @functools.partial(
      pl.kernel,
      mesh=mesh,
      out_type=(
          jax.ShapeDtypeStruct((NW, SUMW), jnp.float32),
          jax.ShapeDtypeStruct((NW, CNTW), jnp.float32),
      ),
      scratch_types=[
          pltpu.VMEM((CHUNK,), jnp.int32),
          pltpu.VMEM((CHUNK, D), jnp.float32),
          pltpu.VMEM((SUMW,), jnp.float32),
          pltpu.VMEM((CNTW,), jnp.float32),
      ],
  )
  def k(x_hbm, b_hbm, zs_hbm, zc_hbm, sums_hbm, cnts_hbm,
        idx_v, xbuf_v, acc_v, cnt_v):
    c = lax.axis_index("c")
    s = lax.axis_index("s")
    w = c * NS + s
    pltpu.sync_copy(zs_hbm, acc_v)
    pltpu.sync_copy(zc_hbm, cnt_v)

    one16 = jnp.ones((16,), jnp.float32)
    lo = w * NBLK // NW
    hi = (w + 1) * NBLK // NW

    def block_body(b, carry):
      base = b * CHUNK
      pltpu.sync_copy(b_hbm.at[pl.ds(base, CHUNK)], idx_v)
      pltpu.sync_copy(x_hbm.at[pl.ds(base, CHUNK)], xbuf_v)

      def group_body(g, carry2):
        segs = idx_v[pl.ds(g * 16, 16)]
        for jj in range(16):
          seg = segs[jj]
          off = seg * D
          r = g * 16 + jj
          for j in range(D // 16):
            v = xbuf_v[r, pl.ds(j * 16, 16)]
            plsc.addupdate(acc_v.at[pl.ds(off + j * 16, 16)], v)
          plsc.addupdate(cnt_v.at[pl.ds(seg * 16, 16)], one16)
        return carry2

      lax.fori_loop(0, CHUNK // 16, group_body, carry)
      return carry

    lax.fori_loop(lo, hi, block_body, 0)

    pltpu.sync_copy(acc_v, sums_hbm.at[w])
    pltpu.sync_copy(cnt_v, cnts_hbm.at[w])

  return k(x, batch, zsum, zcnt)


def _combine(sums, cnts):
  def body(s_ref, c_ref, o_ref):
    ssum = jnp.sum(s_ref[...], axis=0)
    cc = jnp.sum(c_ref[...][:, :, 0:1], axis=0)
    o_ref[...] = ssum / jnp.maximum(cc, 1.0)

  return pl.pallas_call(
      body,
      out_shape=jax.ShapeDtypeStruct((NSEG, D), jnp.float32),
  )(sums, cnts)


@jax.jit
def kernel(x, batch):
  batch = batch.astype(jnp.int32)
  zsum = jnp.zeros((SUMW,), jnp.float32)
  zcnt = jnp.zeros((CNTW,), jnp.float32)
  sums, cnts = _sc_segment_sums(x, batch, zsum, zcnt)
  return _combine(sums.reshape(NW, NSEG, D), cnts.reshape(NW, NSEG, 16))


# speedup vs baseline: 4.4697x; 1.5447x over previous
import functools

import jax
import jax.numpy as jnp
from jax import lax
from jax.experimental import pallas as pl
from jax.experimental.pallas import tpu as pltpu
from jax.experimental.pallas import tpu_sc as plsc

N_ROWS = 50000
D = 256
NSEG = 128
CHUNK = 80
NBLK = N_ROWS // CHUNK
NC = 2
NS = 16
NW = NC * NS
SUMW = NSEG * D
CNTW = NSEG * 16


def _sc_segment_sums(x, batch, zsum, zcnt):
  mesh = plsc.VectorSubcoreMesh(core_axis_name="c", subcore_axis_name="s")

  @functools.partial(
      pl.kernel,
      mesh=mesh,
      out_type=(
          jax.ShapeDtypeStruct((NW, SUMW), jnp.float32),
          jax.ShapeDtypeStruct((NW, CNTW), jnp.float32),
      ),
      scratch_types=[
          pltpu.VMEM((CHUNK,), jnp.int32),
          pltpu.VMEM((CHUNK, D), jnp.float32),
          pltpu.VMEM((SUMW,), jnp.float32),
          pltpu.VMEM((CNTW,), jnp.float32),
      ],
  )
  def k(x_hbm, b_hbm, zs_hbm, zc_hbm, sums_hbm, cnts_hbm,
        idx_v, xbuf_v, acc_v, cnt_v):
    c = lax.axis_index("c")
    s = lax.axis_index("s")
    w = c * NS + s
    pltpu.sync_copy(zs_hbm, acc_v)
    pltpu.sync_copy(zc_hbm, cnt_v)

    one16 = jnp.ones((16,), jnp.float32)
    lo = w * NBLK // NW
    hi = (w + 1) * NBLK // NW

    def block_body(b, carry):
      base = b * CHUNK
      pltpu.sync_copy(b_hbm.at[pl.ds(base, CHUNK)], idx_v)
      pltpu.sync_copy(x_hbm.at[pl.ds(base, CHUNK)], xbuf_v)

      def group_body(g, carry2):
        segs = idx_v[pl.ds(g * 16, 16)]
        for jj in range(16):
          seg = segs[jj]
          off = seg * D
          r = g * 16 + jj
          vs = [xbuf_v[r, pl.ds(j * 16, 16)] for j in range(D // 16)]
          for j in range(D // 16):
            plsc.addupdate(acc_v.at[pl.ds(off + j * 16, 16)], vs[j])
          plsc.addupdate(cnt_v.at[pl.ds(seg * 16, 16)], one16)
        return carry2

      lax.fori_loop(0, CHUNK // 16, group_body, carry)
      return carry

    lax.fori_loop(lo, hi, block_body, 0)

    pltpu.sync_copy(acc_v, sums_hbm.at[w])
    pltpu.sync_copy(cnt_v, cnts_hbm.at[w])

  return k(x, batch, zsum, zcnt)


def _combine(sums, cnts):
  def body(s_ref, c_ref, o_ref):
    ssum = jnp.sum(s_ref[...], axis=0)
    cc = jnp.sum(c_ref[...][:, :, 0:1], axis=0)
    o_ref[...] = ssum / jnp.maximum(cc, 1.0)

  return pl.pallas_call(
      body,
      out_shape=jax.ShapeDtypeStruct((NSEG, D), jnp.float32),
  )(sums, cnts)


@jax.jit
def kernel(x, batch):
  batch = batch.astype(jnp.int32)
  zsum = jnp.zeros((SUMW,), jnp.float32)
  zcnt = jnp.zeros((CNTW,), jnp.float32)
  sums, cnts = _sc_segment_sums(x, batch, zsum, zcnt)
  return _combine(sums.reshape(NW, NSEG, D), cnts.reshape(NW, NSEG, 16))


# speedup vs baseline: 6.5249x; 1.4598x over previous
import functools

import jax
import jax.numpy as jnp
from jax import lax
from jax.experimental import pallas as pl
from jax.experimental.pallas import tpu as pltpu
from jax.experimental.pallas import tpu_sc as plsc

N_ROWS = 50000
D = 256
NSEG = 128
CHUNK = 80
NBLK = N_ROWS // CHUNK
NC = 2
NS = 16
NW = NC * NS
SUMW = NSEG * D
CNTW = NSEG * 16


def _sc_segment_sums(x, batch, zsum, zcnt):
  mesh = plsc.VectorSubcoreMesh(core_axis_name="c", subcore_axis_name="s")

  @functools.partial(
      pl.kernel,
      mesh=mesh,
      out_type=(
          jax.ShapeDtypeStruct((NW, SUMW), jnp.float32),
          jax.ShapeDtypeStruct((NW, CNTW), jnp.float32),
      ),
      scratch_types=[
          pltpu.VMEM((CHUNK,), jnp.int32),
          pltpu.VMEM((CHUNK,), jnp.int32),
          pltpu.VMEM((CHUNK, D), jnp.float32),
          pltpu.VMEM((CHUNK, D), jnp.float32),
          pltpu.VMEM((SUMW,), jnp.float32),
          pltpu.VMEM((CNTW,), jnp.float32),
          pltpu.SemaphoreType.DMA,
          pltpu.SemaphoreType.DMA,
      ],
  )
  def k(x_hbm, b_hbm, zs_hbm, zc_hbm, sums_hbm, cnts_hbm,
        idx0, idx1, xb0, xb1, acc_v, cnt_v, sem0, sem1):
    c = lax.axis_index("c")
    s = lax.axis_index("s")
    w = c * NS + s
    pltpu.sync_copy(zs_hbm, acc_v)
    pltpu.sync_copy(zc_hbm, cnt_v)

    one16 = jnp.ones((16,), jnp.float32)
    lo = w * NBLK // NW
    hi = (w + 1) * NBLK // NW

    def issue(b, idxbuf, xbuf, sem):
      base = b * CHUNK
      pltpu.async_copy(b_hbm.at[pl.ds(base, CHUNK)], idxbuf, sem)
      pltpu.async_copy(x_hbm.at[pl.ds(base, CHUNK)], xbuf, sem)

    def drain(idxbuf, xbuf, sem):
      pltpu.make_async_copy(b_hbm.at[pl.ds(0, CHUNK)], idxbuf, sem).wait()
      pltpu.make_async_copy(x_hbm.at[pl.ds(0, CHUNK)], xbuf, sem).wait()

    def compute(idxbuf, xbuf):
      def group_body(g, carry2):
        segs = idxbuf[pl.ds(g * 16, 16)]
        for jj in range(0, 16, 2):
          sa = segs[jj]
          sb = segs[jj + 1]
          offa = sa * D
          offb = sb * D
          ra = g * 16 + jj
          rb = ra + 1
          va = [xbuf[ra, pl.ds(j * 16, 16)] for j in range(D // 16)]
          vb = [xbuf[rb, pl.ds(j * 16, 16)] for j in range(D // 16)]
          for j in range(D // 16):
            plsc.addupdate(acc_v.at[pl.ds(offa + j * 16, 16)], va[j])
          plsc.addupdate(cnt_v.at[pl.ds(sa * 16, 16)], one16)
          for j in range(D // 16):
            plsc.addupdate(acc_v.at[pl.ds(offb + j * 16, 16)], vb[j])
          plsc.addupdate(cnt_v.at[pl.ds(sb * 16, 16)], one16)
        return carry2

      lax.fori_loop(0, CHUNK // 16, group_body, 0)

    issue(lo, idx0, xb0, sem0)

    def pair_body(q, carry):
      b0 = lo + 2 * q
      b1 = b0 + 1

      @pl.when(b0 < hi)
      def _():
        drain(idx0, xb0, sem0)

        @pl.when(b1 < hi)
        def _():
          issue(b1, idx1, xb1, sem1)

        compute(idx0, xb0)

      @pl.when(b1 < hi)
      def _():
        drain(idx1, xb1, sem1)

        @pl.when(b1 + 1 < hi)
        def _():
          issue(b1 + 1, idx0, xb0, sem0)

        compute(idx1, xb1)

      return carry

    max_pairs = (NBLK // NW + 2) // 2
    lax.fori_loop(0, max_pairs, pair_body, 0)

    pltpu.sync_copy(acc_v, sums_hbm.at[w])
    pltpu.sync_copy(cnt_v, cnts_hbm.at[w])

  return k(x, batch, zsum, zcnt)


def _combine(sums, cnts):
  def body(s_ref, c_ref, o_ref):
    ssum = jnp.sum(s_ref[...], axis=0)
    cc = jnp.sum(c_ref[...][:, :, 0:1], axis=0)
    o_ref[...] = ssum / jnp.maximum(cc, 1.0)

  return pl.pallas_call(
      body,
      out_shape=jax.ShapeDtypeStruct((NSEG, D), jnp.float32),
  )(sums, cnts)


@jax.jit
def kernel(x, batch):
  batch = batch.astype(jnp.int32)
  zsum = jnp.zeros((SUMW,), jnp.float32)
  zcnt = jnp.zeros((CNTW,), jnp.float32)
  sums, cnts = _sc_segment_sums(x, batch, zsum, zcnt)
  return _combine(sums.reshape(NW, NSEG, D), cnts.reshape(NW, NSEG, 16))
